# Initial kernel scaffold; baseline (speedup 1.0000x reference)
import jax
import jax.numpy as jnp
from jax.experimental import pallas as pl


def kernel(x_bus, x_gen, lin_W, lin_b, kW, kb, qW, qb, vW, vb, aW, ab, skip, a_rel, m_rel, p_rel, fW, fb, bn_g, bn_b, fW_out, fb_out, edge_index_bb, edge_index_gb, edge_index_bg, node_idx, num_graphs):
    raise NotImplementedError("write your pallas kernel here")


# speedup vs baseline: 1.0456x; 1.0456x over previous
import functools
import jax
import jax.numpy as jnp
import numpy as np
from jax import lax
from jax.experimental import pallas as pl
from jax.experimental.pallas import tpu as pltpu

H = 4
D = 16
HID = 64
SQRT_D = 4.0
BN_SCALE = 1.0 / float(np.sqrt(1.0 + 1e-5))


def _matmul_kernel(x_ref, w_ref, b_ref, o_ref, *, relu):
    o = jnp.dot(x_ref[...], w_ref[...], preferred_element_type=jnp.float32)
    o = o + b_ref[...]
    if relu:
        o = jax.nn.relu(o)
    o_ref[...] = o


def _matmul(x, W, b, relu=False, blk=1024):
    N, K = x.shape
    C = W.shape[1]
    return pl.pallas_call(
        functools.partial(_matmul_kernel, relu=relu),
        grid=(pl.cdiv(N, blk),),
        in_specs=[
            pl.BlockSpec((blk, K), lambda i: (i, 0)),
            pl.BlockSpec((K, C), lambda i: (0, 0)),
            pl.BlockSpec((1, C), lambda i: (0, 0)),
        ],
        out_specs=pl.BlockSpec((blk, C), lambda i: (i, 0)),
        out_shape=jax.ShapeDtypeStruct((N, C), jnp.float32),
    )(x, W, b.reshape(1, C))


def _combine_kernel(*refs, ntypes):
    if ntypes == 2:
        n1, d1, n2, d2, x_ref, aW_ref, ab_ref, sk_ref, o_ref = refs
        o = jnp.minimum(n1[...] / (d1[...] + 1e-16), n2[...] / (d2[...] + 1e-16))
    else:
        n1, d1, x_ref, aW_ref, ab_ref, sk_ref, o_ref = refs
        o = n1[...] / (d1[...] + 1e-16)
    o = jax.nn.gelu(o)
    o = jnp.dot(o, aW_ref[...], preferred_element_type=jnp.float32) + ab_ref[...]
    sk = sk_ref[0, 0]
    o_ref[...] = sk * o + (1.0 - sk) * x_ref[...]


def _combine(nums_dens, x_old, aW, ab, sk, blk=1024):
    N = x_old.shape[0]
    ntypes = len(nums_dens)
    args = []
    for num, den in nums_dens:
        args += [num, den]
    args += [x_old, aW, ab.reshape(1, HID), sk.reshape(1, 1)]
    nd_specs = []
    for _ in range(ntypes):
        nd_specs += [pl.BlockSpec((blk, HID), lambda i: (i, 0)),
                     pl.BlockSpec((blk, HID), lambda i: (i, 0))]
    return pl.pallas_call(
        functools.partial(_combine_kernel, ntypes=ntypes),
        grid=(pl.cdiv(N, blk),),
        in_specs=nd_specs + [
            pl.BlockSpec((blk, HID), lambda i: (i, 0)),
            pl.BlockSpec((HID, HID), lambda i: (0, 0)),
            pl.BlockSpec((1, HID), lambda i: (0, 0)),
            pl.BlockSpec((1, 1), lambda i: (0, 0), memory_space=pltpu.SMEM),
        ],
        out_specs=pl.BlockSpec((blk, HID), lambda i: (i, 0)),
        out_shape=jax.ShapeDtypeStruct((N, HID), jnp.float32),
    )(*args)


def _mlp_kernel(h_ref, fW_ref, fb_ref, g_ref, b_ref, wo_ref, bo_ref, o_ref):
    h = h_ref[...]
    for i in range(4):
        h = jnp.dot(h, fW_ref[i], preferred_element_type=jnp.float32) + fb_ref[i]
        h = h * (g_ref[i] * BN_SCALE) + b_ref[i]
        h = jax.nn.relu(h)
    o_ref[...] = jnp.dot(h, wo_ref[...], preferred_element_type=jnp.float32) + bo_ref[...]


def _mlp(h, fW, fb, bn_g, bn_b, fW_out, fb_out, blk=2048):
    N = h.shape[0]
    return pl.pallas_call(
        _mlp_kernel,
        grid=(pl.cdiv(N, blk),),
        in_specs=[
            pl.BlockSpec((blk, HID), lambda i: (i, 0)),
            pl.BlockSpec((4, HID, HID), lambda i: (0, 0, 0)),
            pl.BlockSpec((4, 1, HID), lambda i: (0, 0, 0)),
            pl.BlockSpec((4, 1, HID), lambda i: (0, 0, 0)),
            pl.BlockSpec((4, 1, HID), lambda i: (0, 0, 0)),
            pl.BlockSpec((HID, 1), lambda i: (0, 0)),
            pl.BlockSpec((1, 1), lambda i: (0, 0)),
        ],
        out_specs=pl.BlockSpec((blk, 1), lambda i: (i, 0)),
        out_shape=jax.ShapeDtypeStruct((N, 1), jnp.float32),
    )(h, fW, fb.reshape(4, 1, HID), bn_g.reshape(4, 1, HID),
      bn_b.reshape(4, 1, HID), fW_out, fb_out.reshape(1, 1))


def _edge_phase(q_dst, kv_src, row, col, n_dst, p_scaled):
    kvr = kv_src.reshape(-1, H, 2, D)
    k_rel = kvr[:, :, 0]
    v_rel = kvr[:, :, 1]
    qd = q_dst.reshape(-1, H, D)
    alpha = (qd[col] * k_rel[row]).sum(-1) * p_scaled
    e = jnp.exp(alpha)
    num = jax.ops.segment_sum(v_rel[row] * e[:, :, None], col, num_segments=n_dst)
    den = jax.ops.segment_sum(e, col, num_segments=n_dst)
    num = num.reshape(n_dst, HID)
    den64 = jnp.repeat(den, D, axis=1)
    return num, den64


def _fold_kv(kW_li, kb_li, vW_li, vb_li, a_rel_lj, m_rel_lj):
    Wk = kW_li.reshape(HID, H, D)
    Wv = vW_li.reshape(HID, H, D)
    Wk_rel = jnp.einsum('chd,hde->hce', Wk, a_rel_lj)
    Wv_rel = jnp.einsum('chd,hde->hce', Wv, m_rel_lj)
    bk_rel = jnp.einsum('hd,hde->he', kb_li.reshape(H, D), a_rel_lj)
    bv_rel = jnp.einsum('hd,hde->he', vb_li.reshape(H, D), m_rel_lj)
    Wcat = jnp.stack([Wk_rel, Wv_rel], axis=2)
    Wcat = jnp.stack([Wk_rel, Wv_rel], axis=1)
    Wcat = Wcat.transpose(2, 0, 1, 3).reshape(HID, H * 2 * D)
    bcat = jnp.stack([bk_rel, bv_rel], axis=1).reshape(H * 2 * D)
    return Wcat, bcat


def kernel(x_bus, x_gen, lin_W, lin_b, kW, kb, qW, qb, vW, vb, aW, ab, skip,
           a_rel, m_rel, p_rel, fW, fb, bn_g, bn_b, fW_out, fb_out,
           edge_index_bb, edge_index_gb, edge_index_bg, node_idx, num_graphs):
    xb = _matmul(x_bus, lin_W[0], lin_b[0], relu=True)
    xg = _matmul(x_gen, lin_W[1], lin_b[1], relu=True)
    x = {'bus': xb, 'gen': xg}
    sk_all = jax.nn.sigmoid(skip)
    edges = [('bus', 'bus', edge_index_bb), ('gen', 'bus', edge_index_gb),
             ('bus', 'gen', edge_index_bg)]
    ntype = {'bus': 0, 'gen': 1}

    for l in range(2):
        Wkv = {}
        bkv = {}
        for j in range(3):
            i = ntype[edges[j][0]]
            Wkv[j], bkv[j] = _fold_kv(kW[l, i], kb[l, i], vW[l, i], vb[l, i],
                                      a_rel[l, j], m_rel[l, j])
        W_bus = jnp.concatenate([qW[l, 0], Wkv[0], Wkv[2]], axis=1)
        b_bus = jnp.concatenate([qb[l, 0], bkv[0], bkv[2]])
        W_gen = jnp.concatenate([qW[l, 1], Wkv[1]], axis=1)
        b_gen = jnp.concatenate([qb[l, 1], bkv[1]])
        P_bus = _matmul(x['bus'], W_bus, b_bus)
        P_gen = _matmul(x['gen'], W_gen, b_gen)
        q = {'bus': P_bus[:, :HID], 'gen': P_gen[:, :HID]}
        kv = {0: P_bus[:, HID:HID + 2 * HID], 1: P_gen[:, HID:],
              2: P_bus[:, HID + 2 * HID:]}

        acc = {'bus': [], 'gen': []}
        for j, (src, dst, ei) in enumerate(edges):
            n_dst = x[dst].shape[0]
            num, den64 = _edge_phase(q[dst], kv[j], ei[0], ei[1], n_dst,
                                     p_rel[l, j] / SQRT_D)
            acc[dst].append((num, den64))
        newx = {}
        for nt in ('bus', 'gen'):
            i = ntype[nt]
            newx[nt] = _combine(acc[nt], x[nt], aW[l, i], ab[l, i], sk_all[l, i])
        x = newx

    G = x_bus.shape[0] // 19
    idxs = (node_idx[0][None, :] + 19 * jnp.arange(G)[:, None]).reshape(-1)
    h = x['bus'][idxs]
    return _mlp(h, fW, fb, bn_g, bn_b, fW_out, fb_out)


# speedup vs baseline: 39.0841x; 37.3788x over previous
import functools
import jax
import jax.numpy as jnp
import numpy as np
from jax import lax
from jax.experimental import pallas as pl
from jax.experimental.pallas import tpu as pltpu
from jax.experimental.pallas import tpu_sc as plsc

H = 4
D = 16
HID = 64
SQRT_D = 4.0
BN_SCALE = 1.0 / float(np.sqrt(1.0 + 1e-5))
W = 128
_GD = lax.GatherDimensionNumbers(offset_dims=(), collapsed_slice_dims=(0,),
                                 start_index_map=(0,))


def _shuffle(v, idx):
    return lax.gather(v, idx[:, None], _GD, (1,),
                      mode=lax.GatherScatterMode.PROMISE_IN_BOUNDS)
NW = 32
ZR = 128


def _matmul_kernel(x_ref, w_ref, b_ref, o_ref, *, relu):
    o = jnp.dot(x_ref[...], w_ref[...], preferred_element_type=jnp.float32)
    o = o + b_ref[...]
    if relu:
        o = jax.nn.relu(o)
    o_ref[...] = o


def _matmul(x, Wm, b, relu=False, blk=1024):
    N, K = x.shape
    C = Wm.shape[1]
    return pl.pallas_call(
        functools.partial(_matmul_kernel, relu=relu),
        grid=(pl.cdiv(N, blk),),
        in_specs=[
            pl.BlockSpec((blk, K), lambda i: (i, 0)),
            pl.BlockSpec((K, C), lambda i: (0, 0)),
            pl.BlockSpec((1, C), lambda i: (0, 0)),
        ],
        out_specs=pl.BlockSpec((blk, C), lambda i: (i, 0)),
        out_shape=jax.ShapeDtypeStruct((N, C), jnp.float32),
    )(x, Wm, b.reshape(1, C))


def _combine_kernel(*refs, ntypes):
    def norm(A):
        parts = []
        for h in range(H):
            num = A[2 * h, :, 0:16] + A[2 * h + 1, :, 0:16]
            den = A[2 * h, :, 16:17] + A[2 * h + 1, :, 16:17]
            parts.append(num / (den + 1e-16))
        return jnp.concatenate(parts, axis=1)

    if ntypes == 2:
        A1, A2, x_ref, aW_ref, ab_ref, sk_ref, o_ref = refs
        o = jnp.minimum(norm(A1[...]), norm(A2[...]))
    else:
        A1, x_ref, aW_ref, ab_ref, sk_ref, o_ref = refs
        o = norm(A1[...])
    o = jax.nn.gelu(o)
    o = jnp.dot(o, aW_ref[...], preferred_element_type=jnp.float32) + ab_ref[...]
    sk = sk_ref[0, 0]
    o_ref[...] = sk * o + (1.0 - sk) * x_ref[...]


def _combine(accs, x_old, aW, ab, sk, blk=1024):
    N = x_old.shape[0]
    ntypes = len(accs)
    args = list(accs) + [x_old, aW, ab.reshape(1, HID), sk.reshape(1, 1)]
    a_specs = [pl.BlockSpec((2 * H, blk, 32), lambda i: (0, i, 0))
               for _ in range(ntypes)]
    return pl.pallas_call(
        functools.partial(_combine_kernel, ntypes=ntypes),
        grid=(pl.cdiv(N, blk),),
        in_specs=a_specs + [
            pl.BlockSpec((blk, HID), lambda i: (i, 0)),
            pl.BlockSpec((HID, HID), lambda i: (0, 0)),
            pl.BlockSpec((1, HID), lambda i: (0, 0)),
            pl.BlockSpec((1, 1), lambda i: (0, 0), memory_space=pltpu.SMEM),
        ],
        out_specs=pl.BlockSpec((blk, HID), lambda i: (i, 0)),
        out_shape=jax.ShapeDtypeStruct((N, HID), jnp.float32),
    )(*args)


def _mlp_kernel(h_ref, fW_ref, fb_ref, g_ref, b_ref, wo_ref, bo_ref, o_ref):
    h = h_ref[...]
    for i in range(4):
        h = jnp.dot(h, fW_ref[i], preferred_element_type=jnp.float32) + fb_ref[i]
        h = h * (g_ref[i] * BN_SCALE) + b_ref[i]
        h = jax.nn.relu(h)
    o_ref[...] = jnp.dot(h, wo_ref[...], preferred_element_type=jnp.float32) + bo_ref[...]


def _mlp(h, fW, fb, bn_g, bn_b, fW_out, fb_out, blk=2048):
    N = h.shape[0]
    return pl.pallas_call(
        _mlp_kernel,
        grid=(pl.cdiv(N, blk),),
        in_specs=[
            pl.BlockSpec((blk, HID), lambda i: (i, 0)),
            pl.BlockSpec((4, HID, HID), lambda i: (0, 0, 0)),
            pl.BlockSpec((4, 1, HID), lambda i: (0, 0, 0)),
            pl.BlockSpec((4, 1, HID), lambda i: (0, 0, 0)),
            pl.BlockSpec((4, 1, HID), lambda i: (0, 0, 0)),
            pl.BlockSpec((HID, 1), lambda i: (0, 0)),
            pl.BlockSpec((1, 1), lambda i: (0, 0)),
        ],
        out_specs=pl.BlockSpec((blk, 1), lambda i: (i, 0)),
        out_shape=jax.ShapeDtypeStruct((N, 1), jnp.float32),
    )(h, fW, fb.reshape(4, 1, HID), bn_g.reshape(4, 1, HID),
      bn_b.reshape(4, 1, HID), fW_out, fb_out.reshape(1, 1))


@functools.lru_cache(maxsize=None)
def _make_edge_kernel(n_src, nacc, nwin):
    mesh = plsc.VectorSubcoreMesh(core_axis_name="c", subcore_axis_name="s")
    stripe = nacc // 16
    chunk = nwin * W
    out_type = jax.ShapeDtypeStruct((2 * H, nacc, 32), jnp.float32)
    scratch = [
        pltpu.VMEM_SHARED((nacc, 32), jnp.float32),
        pltpu.VMEM((W,), jnp.int32),
        pltpu.VMEM((W,), jnp.int32),
        pltpu.VMEM((W,), jnp.int32),
        pltpu.VMEM((W,), jnp.int32),
        pltpu.VMEM((W, 16), jnp.float32),
        pltpu.VMEM((W, 32), jnp.float32),
        pltpu.VMEM((W, 32), jnp.float32),
        pltpu.VMEM((16,), jnp.float32),
        pltpu.VMEM((ZR, 32), jnp.float32),
        pltpu.SemaphoreType.DMA,
        pltpu.SemaphoreType.DMA,
    ]

    @functools.partial(pl.kernel, out_type=out_type, mesh=mesh,
                       scratch_types=scratch,
                       compiler_params=pltpu.CompilerParams(
                           use_tc_tiling_on_sc=False))
    def k(q4, kv8, rowp, colp, psc, out, acc, rowv, colv, gq, gkv, qbuf,
          kvbuf, upd, pbuf, zbuf, semq, semkv):
        c = lax.axis_index("c")
        s = lax.axis_index("s")
        wid = s * 2 + c
        z16 = jnp.zeros((16,), jnp.float32)
        iota16 = lax.iota(jnp.int32, 16)

        def zrow(r, _):
            zbuf[r, 0:16] = z16
            zbuf[r, 16:32] = z16
            return 0
        lax.fori_loop(0, ZR, zrow, 0)

        pltpu.sync_copy(psc, pbuf)
        pvec = pbuf[...]
        base = s * stripe

        for h in range(H):
            def zstripe(z, _):
                pltpu.sync_copy(zbuf, acc.at[pl.ds(base + z * ZR, ZR)])
                return 0
            lax.fori_loop(0, stripe // ZR, zstripe, 0)
            plsc.subcore_barrier()
            p_h = pvec[h]

            l0 = iota16 == 0
            shuf_idx = [jnp.bitwise_xor(iota16, d) for d in (1, 2, 4, 8)]

            def window(win, _):
                ebase = wid * chunk + win * W
                pltpu.sync_copy(rowp.at[pl.ds(ebase, W)], rowv)
                pltpu.sync_copy(colp.at[pl.ds(ebase, W)], colv)
                for g in range(W // 16):
                    cv = colv[pl.ds(g * 16, 16)]
                    rv = rowv[pl.ds(g * 16, 16)]
                    gq[pl.ds(g * 16, 16)] = cv * 4 + h
                    gkv[pl.ds(g * 16, 16)] = rv * 4 + h
                cp_q = pltpu.async_copy(q4.at[gq], qbuf, semq)
                cp_kv = pltpu.async_copy(kv8.at[gkv], kvbuf, semkv)
                cp_q.wait()
                cp_kv.wait()
                for e in range(W):
                    qv = qbuf[e, :]
                    kk = kvbuf[e, 0:16]
                    a = qv * kk
                    for si in shuf_idx:
                        a = a + _shuffle(a, si)
                    ev = jnp.exp(a * p_h)
                    vv = kvbuf[e, 16:32]
                    upd[e, 0:16] = vv * ev
                    upd[e, 16:32] = jnp.where(l0, ev, z16)
                pltpu.sync_copy(upd, acc.at[colv], add=True)
                return 0
            lax.fori_loop(0, nwin, window, 0)
            plsc.subcore_barrier()
            pltpu.sync_copy(acc.at[pl.ds(base, stripe)],
                            out.at[2 * h + c, pl.ds(base, stripe)])
            plsc.subcore_barrier()

    return k


def _edge_sc(q_pad, kv, rowp, colp, psc, nacc, nwin):
    n_src = kv.shape[0]
    k = _make_edge_kernel(n_src, nacc, nwin)
    q4 = q_pad.reshape(nacc * 4, 16)
    kv8 = kv.reshape(n_src * 4, 32)
    return k(q4, kv8, rowp, colp, psc)


@functools.lru_cache(maxsize=None)
def _make_gather_kernel(n_rows, n_out, per_w):
    mesh = plsc.VectorSubcoreMesh(core_axis_name="c", subcore_axis_name="s")
    nchunk = per_w // W
    out_type = jax.ShapeDtypeStruct((n_out, HID), jnp.float32)
    scratch = [
        pltpu.VMEM((nchunk, W), jnp.int32),
        pltpu.VMEM((per_w, HID), jnp.float32),
        pltpu.SemaphoreType.DMA,
    ]

    @functools.partial(pl.kernel, out_type=out_type, mesh=mesh,
                       scratch_types=scratch,
                       compiler_params=pltpu.CompilerParams(
                           use_tc_tiling_on_sc=False))
    def k(x_hbm, idx_hbm, out_hbm, idxv, rows, sem):
        c = lax.axis_index("c")
        s = lax.axis_index("s")
        wid = s * 2 + c
        base = wid * per_w
        pltpu.sync_copy(idx_hbm.at[pl.ds(wid * nchunk, nchunk)], idxv)
        for j in range(nchunk):
            pltpu.async_copy(x_hbm.at[idxv.at[j]],
                             rows.at[pl.ds(j * W, W)], sem).wait()
        pltpu.sync_copy(rows, out_hbm.at[pl.ds(base, per_w)])

    return k


def _gather_rows(x, idx_pad):
    n_out = idx_pad.shape[0]
    per_w = n_out // NW
    k = _make_gather_kernel(x.shape[0], n_out, per_w)
    return k(x, idx_pad.reshape(n_out // W, W))


def _fold_kv(kW_li, kb_li, vW_li, vb_li, a_rel_lj, m_rel_lj):
    Wk = kW_li.reshape(HID, H, D)
    Wv = vW_li.reshape(HID, H, D)
    Wk_rel = jnp.einsum('chd,hde->hce', Wk, a_rel_lj)
    Wv_rel = jnp.einsum('chd,hde->hce', Wv, m_rel_lj)
    bk_rel = jnp.einsum('hd,hde->he', kb_li.reshape(H, D), a_rel_lj)
    bv_rel = jnp.einsum('hd,hde->he', vb_li.reshape(H, D), m_rel_lj)
    Wcat = jnp.stack([Wk_rel, Wv_rel], axis=1)
    Wcat = Wcat.transpose(2, 0, 1, 3).reshape(HID, H * 2 * D)
    bcat = jnp.stack([bk_rel, bv_rel], axis=1).reshape(H * 2 * D)
    return Wcat, bcat


def _pad_to(n, unit):
    return ((n + unit - 1) // unit) * unit


def kernel(x_bus, x_gen, lin_W, lin_b, kW, kb, qW, qb, vW, vb, aW, ab, skip,
           a_rel, m_rel, p_rel, fW, fb, bn_g, bn_b, fW_out, fb_out,
           edge_index_bb, edge_index_gb, edge_index_bg, node_idx, num_graphs):
    n_bus = x_bus.shape[0]
    n_gen = x_gen.shape[0]
    nacc_b = _pad_to(n_bus + 16, 2048)
    nacc_g = _pad_to(n_gen + 16, 2048)

    xb = _matmul(x_bus, lin_W[0], lin_b[0], relu=True)
    xg = _matmul(x_gen, lin_W[1], lin_b[1], relu=True)
    x = {'bus': xb, 'gen': xg}
    sk_all = jax.nn.sigmoid(skip)
    edges = [('bus', 'bus', edge_index_bb), ('gen', 'bus', edge_index_gb),
             ('bus', 'gen', edge_index_bg)]
    ntype = {'bus': 0, 'gen': 1}
    naccs = {'bus': nacc_b, 'gen': nacc_g}

    rowps, colps, nwins = {}, {}, {}
    for j, (src, dst, ei) in enumerate(edges):
        E = ei.shape[1]
        E_pad = _pad_to(E, NW * W)
        nwins[j] = E_pad // (NW * W)
        npad = E_pad - E
        ar = jnp.arange(npad, dtype=jnp.int32)
        n_dst = n_bus if dst == 'bus' else n_gen
        rowps[j] = jnp.concatenate([ei[0].astype(jnp.int32), ar % 64])
        colps[j] = jnp.concatenate([ei[1].astype(jnp.int32), n_dst + ar % 16])

    for l in range(2):
        Wkv, bkv = {}, {}
        for j in range(3):
            i = ntype[edges[j][0]]
            Wkv[j], bkv[j] = _fold_kv(kW[l, i], kb[l, i], vW[l, i], vb[l, i],
                                      a_rel[l, j], m_rel[l, j])
        W_bus = jnp.concatenate([qW[l, 0], Wkv[0], Wkv[2]], axis=1)
        b_bus = jnp.concatenate([qb[l, 0], bkv[0], bkv[2]])
        W_gen = jnp.concatenate([qW[l, 1], Wkv[1]], axis=1)
        b_gen = jnp.concatenate([qb[l, 1], bkv[1]])
        P_bus = _matmul(x['bus'], W_bus, b_bus)
        P_gen = _matmul(x['gen'], W_gen, b_gen)
        qpad = {
            'bus': jnp.pad(P_bus[:, :HID], ((0, nacc_b - n_bus), (0, 0))),
            'gen': jnp.pad(P_gen[:, :HID], ((0, nacc_g - n_gen), (0, 0))),
        }
        kv = {0: P_bus[:, HID:HID + 2 * HID], 1: P_gen[:, HID:],
              2: P_bus[:, HID + 2 * HID:]}

        acc = {'bus': [], 'gen': []}
        for j, (src, dst, ei) in enumerate(edges):
            psc = jnp.zeros((16,), jnp.float32).at[:H].set(p_rel[l, j] / SQRT_D)
            A = _edge_sc(qpad[dst], kv[j], rowps[j], colps[j], psc,
                         naccs[dst], nwins[j])
            acc[dst].append(A)
        newx = {}
        for nt in ('bus', 'gen'):
            i = ntype[nt]
            newx[nt] = _combine(acc[nt], x[nt], aW[l, i], ab[l, i],
                                sk_all[l, i])
        x = newx

    G = n_bus // 19
    idxs = (node_idx[0][None, :] + 19 * jnp.arange(G)[:, None]).reshape(-1)
    n_sel = idxs.shape[0]
    n_sel_pad = _pad_to(n_sel, NW * W)
    ar = jnp.arange(n_sel_pad - n_sel, dtype=jnp.int32)
    idx_pad = jnp.concatenate([idxs.astype(jnp.int32), ar % 64])
    h = _gather_rows(x['bus'], idx_pad)[:n_sel]
    return _mlp(h, fW, fb, bn_g, bn_b, fW_out, fb_out)


# speedup vs baseline: 66.5375x; 1.7024x over previous
import functools
import jax
import jax.numpy as jnp
import numpy as np
from jax import lax
from jax.experimental import pallas as pl
from jax.experimental.pallas import tpu as pltpu
from jax.experimental.pallas import tpu_sc as plsc

H = 4
D = 16
HID = 64
SQRT_D = 4.0
BN_SCALE = 1.0 / float(np.sqrt(1.0 + 1e-5))
W = 128
_GD = lax.GatherDimensionNumbers(offset_dims=(), collapsed_slice_dims=(0,),
                                 start_index_map=(0,))


def _shuffle(v, idx):
    return lax.gather(v, idx[:, None], _GD, (1,),
                      mode=lax.GatherScatterMode.PROMISE_IN_BOUNDS)
NW = 32
ZR = 128


def _matmul_kernel(x_ref, w_ref, b_ref, o_ref, *, relu):
    o = jnp.dot(x_ref[...], w_ref[...], preferred_element_type=jnp.float32)
    o = o + b_ref[...]
    if relu:
        o = jax.nn.relu(o)
    o_ref[...] = o


def _matmul(x, Wm, b, relu=False, blk=1024):
    N, K = x.shape
    C = Wm.shape[1]
    return pl.pallas_call(
        functools.partial(_matmul_kernel, relu=relu),
        grid=(pl.cdiv(N, blk),),
        in_specs=[
            pl.BlockSpec((blk, K), lambda i: (i, 0)),
            pl.BlockSpec((K, C), lambda i: (0, 0)),
            pl.BlockSpec((1, C), lambda i: (0, 0)),
        ],
        out_specs=pl.BlockSpec((blk, C), lambda i: (i, 0)),
        out_shape=jax.ShapeDtypeStruct((N, C), jnp.float32),
    )(x, Wm, b.reshape(1, C))


def _combine_kernel(*refs, ntypes):
    def norm(A):
        parts = []
        for h in range(H):
            num = A[2 * h, :, 0:16] + A[2 * h + 1, :, 0:16]
            den = A[2 * h, :, 16:17] + A[2 * h + 1, :, 16:17]
            parts.append(num / (den + 1e-16))
        return jnp.concatenate(parts, axis=1)

    if ntypes == 2:
        A1, A2, x_ref, aW_ref, ab_ref, sk_ref, o_ref = refs
        o = jnp.minimum(norm(A1[...]), norm(A2[...]))
    else:
        A1, x_ref, aW_ref, ab_ref, sk_ref, o_ref = refs
        o = norm(A1[...])
    o = jax.nn.gelu(o)
    o = jnp.dot(o, aW_ref[...], preferred_element_type=jnp.float32) + ab_ref[...]
    sk = sk_ref[0, 0]
    o_ref[...] = sk * o + (1.0 - sk) * x_ref[...]


def _combine(accs, x_old, aW, ab, sk, blk=1024):
    N = x_old.shape[0]
    ntypes = len(accs)
    args = list(accs) + [x_old, aW, ab.reshape(1, HID), sk.reshape(1, 1)]
    a_specs = [pl.BlockSpec((2 * H, blk, 32), lambda i: (0, i, 0))
               for _ in range(ntypes)]
    return pl.pallas_call(
        functools.partial(_combine_kernel, ntypes=ntypes),
        grid=(pl.cdiv(N, blk),),
        in_specs=a_specs + [
            pl.BlockSpec((blk, HID), lambda i: (i, 0)),
            pl.BlockSpec((HID, HID), lambda i: (0, 0)),
            pl.BlockSpec((1, HID), lambda i: (0, 0)),
            pl.BlockSpec((1, 1), lambda i: (0, 0), memory_space=pltpu.SMEM),
        ],
        out_specs=pl.BlockSpec((blk, HID), lambda i: (i, 0)),
        out_shape=jax.ShapeDtypeStruct((N, HID), jnp.float32),
    )(*args)


def _mlp_kernel(h_ref, fW_ref, fb_ref, g_ref, b_ref, wo_ref, bo_ref, o_ref):
    h = h_ref[...]
    for i in range(4):
        h = jnp.dot(h, fW_ref[i], preferred_element_type=jnp.float32) + fb_ref[i]
        h = h * (g_ref[i] * BN_SCALE) + b_ref[i]
        h = jax.nn.relu(h)
    o_ref[...] = jnp.dot(h, wo_ref[...], preferred_element_type=jnp.float32) + bo_ref[...]


def _mlp(h, fW, fb, bn_g, bn_b, fW_out, fb_out, blk=2048):
    N = h.shape[0]
    return pl.pallas_call(
        _mlp_kernel,
        grid=(pl.cdiv(N, blk),),
        in_specs=[
            pl.BlockSpec((blk, HID), lambda i: (i, 0)),
            pl.BlockSpec((4, HID, HID), lambda i: (0, 0, 0)),
            pl.BlockSpec((4, 1, HID), lambda i: (0, 0, 0)),
            pl.BlockSpec((4, 1, HID), lambda i: (0, 0, 0)),
            pl.BlockSpec((4, 1, HID), lambda i: (0, 0, 0)),
            pl.BlockSpec((HID, 1), lambda i: (0, 0)),
            pl.BlockSpec((1, 1), lambda i: (0, 0)),
        ],
        out_specs=pl.BlockSpec((blk, 1), lambda i: (i, 0)),
        out_shape=jax.ShapeDtypeStruct((N, 1), jnp.float32),
    )(h, fW, fb.reshape(4, 1, HID), bn_g.reshape(4, 1, HID),
      bn_b.reshape(4, 1, HID), fW_out, fb_out.reshape(1, 1))


WE = 128
CPW = WE // W


@functools.lru_cache(maxsize=None)
def _make_edge_kernel(n_src, nacc, nwin):
    mesh = plsc.VectorSubcoreMesh(core_axis_name="c", subcore_axis_name="s")
    stripe = nacc // 16
    out_type = jax.ShapeDtypeStruct((2 * H, nacc, 32), jnp.float32)
    scratch = [
        pltpu.VMEM_SHARED((nacc, 32), jnp.float32),
        [pltpu.VMEM((CPW, W), jnp.int32)] * 2,
        [pltpu.VMEM((CPW, W), jnp.int32)] * 2,
        [pltpu.VMEM((CPW, W), jnp.int32)] * 2,
        [pltpu.VMEM((CPW, W), jnp.int32)] * 2,
        [pltpu.VMEM((CPW, W), jnp.int32)] * 2,
        [pltpu.VMEM((WE, 16), jnp.float32)] * 2,
        [pltpu.VMEM((WE, 32), jnp.float32)] * 2,
        [pltpu.VMEM((WE, 32), jnp.float32)] * 2,
        pltpu.VMEM((16,), jnp.float32),
        pltpu.VMEM((ZR, 32), jnp.float32),
        [pltpu.SemaphoreType.DMA] * 2,
        [pltpu.SemaphoreType.DMA] * 2,
        [pltpu.SemaphoreType.DMA] * 2,
        [pltpu.SemaphoreType.DMA] * 2,
    ]

    @functools.partial(pl.kernel, out_type=out_type, mesh=mesh,
                       scratch_types=scratch,
                       compiler_params=pltpu.CompilerParams(
                           use_tc_tiling_on_sc=False))
    def k(q4, kv8, rowp, colp, psc, out, acc, rowv, colv, gq, gkv, sidx,
          qbuf, kvbuf, upd, pbuf, zbuf, semq, semkv, semidx, semsc):
        c = lax.axis_index("c")
        s = lax.axis_index("s")
        wid = s * 2 + c
        z16 = jnp.zeros((16,), jnp.float32)
        iota16 = lax.iota(jnp.int32, 16)
        l0 = iota16 == 0
        shuf_idx = [jnp.bitwise_xor(iota16, d) for d in (1, 2, 4, 8)]

        def zrow(r, _):
            zbuf[r, 0:16] = z16
            zbuf[r, 16:32] = z16
            return 0
        lax.fori_loop(0, ZR, zrow, 0)

        pltpu.sync_copy(psc, pbuf)
        pvec = pbuf[...]
        base = s * stripe

        def idx_slices(w, b):
            off = (wid * nwin + w) * CPW
            return [(rowp.at[pl.ds(off, CPW)], rowv[b]),
                    (colp.at[pl.ds(off, CPW)], colv[b])]

        def idx_sync(w, b):
            for src, dst in idx_slices(w, b):
                pltpu.sync_copy(src, dst)

        def idx_async(w, b):
            for src, dst in idx_slices(w, b):
                pltpu.async_copy(src, dst, semidx[b])

        def idx_drain(w, b):
            for src, dst in idx_slices(w, b):
                pltpu.make_async_copy(src, dst, semidx[b]).wait()

        def build(b, h):
            for j in range(CPW):
                for g in range(W // 16):
                    sl = pl.ds(g * 16, 16)
                    gq[b][j, sl] = colv[b][j, sl] * 4 + h
                    gkv[b][j, sl] = rowv[b][j, sl] * 4 + h

        def fire(b):
            for j in range(CPW):
                pltpu.async_copy(q4.at[gq[b].at[j]],
                                 qbuf[b].at[pl.ds(j * W, W)], semq[b])
                pltpu.async_copy(kv8.at[gkv[b].at[j]],
                                 kvbuf[b].at[pl.ds(j * W, W)], semkv[b])

        def drain(b):
            for j in range(CPW):
                pltpu.make_async_copy(q4.at[gq[b].at[j]],
                                      qbuf[b].at[pl.ds(j * W, W)],
                                      semq[b]).wait()
                pltpu.make_async_copy(kv8.at[gkv[b].at[j]],
                                      kvbuf[b].at[pl.ds(j * W, W)],
                                      semkv[b]).wait()

        def save_sidx(b):
            for j in range(CPW):
                for g in range(W // 16):
                    sl = pl.ds(g * 16, 16)
                    sidx[b][j, sl] = colv[b][j, sl]

        def scat_fire(b):
            for j in range(CPW):
                pltpu.async_copy(upd[b].at[pl.ds(j * W, W)],
                                 acc.at[sidx[b].at[j]], semsc[b], add=True)

        def scat_wait(b):
            for j in range(CPW):
                pltpu.make_async_copy(upd[b].at[pl.ds(j * W, W)],
                                      acc.at[sidx[b].at[j]],
                                      semsc[b]).wait()

        def compute(b, p_h):
            qb, kvb, ub = qbuf[b], kvbuf[b], upd[b]

            def sub(ci, _):
                for jj in range(W):
                    e = ci * W + jj
                    qv = qb[e, :]
                    kk = kvb[e, 0:16]
                    vv = kvb[e, 16:32]
                    a = qv * kk
                    for si in shuf_idx:
                        a = a + _shuffle(a, si)
                    ev = jnp.exp(a * p_h)
                    ub[e, 0:16] = vv * ev
                    ub[e, 16:32] = jnp.where(l0, ev, z16)
                return 0
            lax.fori_loop(0, CPW, sub, 0)

        def head_body(h, _):
            def zstripe(z, _z):
                pltpu.sync_copy(zbuf, acc.at[pl.ds(base + z * ZR, ZR)])
                return 0
            lax.fori_loop(0, stripe // ZR, zstripe, 0)
            plsc.subcore_barrier()
            p_h = _shuffle(pvec, jnp.broadcast_to(h, (16,)))

            idx_sync(0, 0)
            build(0, h)
            fire(0)
            idx_async(1, 1)

            def window(w, t, b, nb):
                @pl.when(w + 1 < nwin)
                def _a():
                    idx_drain(w + 1, nb)
                    build(nb, h)
                    fire(nb)

                @pl.when(t > 0)
                def _b():
                    scat_wait(b)
                save_sidx(b)

                @pl.when(w + 2 < nwin)
                def _c():
                    idx_async(w + 2, b)
                drain(b)
                compute(b, p_h)
                scat_fire(b)

            def pair(t, _t):
                window(2 * t, t, 0, 1)
                window(2 * t + 1, t, 1, 0)
                return 0
            lax.fori_loop(0, nwin // 2, pair, 0)
            scat_wait(0)
            scat_wait(1)
            plsc.subcore_barrier()
            pltpu.sync_copy(acc.at[pl.ds(base, stripe)],
                            out.at[2 * h + c, pl.ds(base, stripe)])
            plsc.subcore_barrier()
            return 0

        lax.fori_loop(0, H, head_body, 0)

    return k


def _edge_sc(q_pad, kv, rowp, colp, psc, nacc, nwin):
    n_src = kv.shape[0]
    k = _make_edge_kernel(n_src, nacc, nwin)
    q4 = q_pad.reshape(nacc * 4, 16)
    kv8 = kv.reshape(n_src * 4, 32)
    return k(q4, kv8, rowp.reshape(-1, W), colp.reshape(-1, W), psc)


@functools.lru_cache(maxsize=None)
def _make_gather_kernel(n_rows, n_out, per_w):
    mesh = plsc.VectorSubcoreMesh(core_axis_name="c", subcore_axis_name="s")
    nchunk = per_w // W
    out_type = jax.ShapeDtypeStruct((n_out, HID), jnp.float32)
    scratch = [
        pltpu.VMEM((nchunk, W), jnp.int32),
        pltpu.VMEM((per_w, HID), jnp.float32),
        pltpu.SemaphoreType.DMA,
    ]

    @functools.partial(pl.kernel, out_type=out_type, mesh=mesh,
                       scratch_types=scratch,
                       compiler_params=pltpu.CompilerParams(
                           use_tc_tiling_on_sc=False))
    def k(x_hbm, idx_hbm, out_hbm, idxv, rows, sem):
        c = lax.axis_index("c")
        s = lax.axis_index("s")
        wid = s * 2 + c
        base = wid * per_w
        pltpu.sync_copy(idx_hbm.at[pl.ds(wid * nchunk, nchunk)], idxv)
        for j in range(nchunk):
            pltpu.async_copy(x_hbm.at[idxv.at[j]],
                             rows.at[pl.ds(j * W, W)], sem).wait()
        pltpu.sync_copy(rows, out_hbm.at[pl.ds(base, per_w)])

    return k


def _gather_rows(x, idx_pad):
    n_out = idx_pad.shape[0]
    per_w = n_out // NW
    k = _make_gather_kernel(x.shape[0], n_out, per_w)
    return k(x, idx_pad.reshape(n_out // W, W))


def _fold_kv(kW_li, kb_li, vW_li, vb_li, a_rel_lj, m_rel_lj):
    Wk = kW_li.reshape(HID, H, D)
    Wv = vW_li.reshape(HID, H, D)
    Wk_rel = jnp.einsum('chd,hde->hce', Wk, a_rel_lj)
    Wv_rel = jnp.einsum('chd,hde->hce', Wv, m_rel_lj)
    bk_rel = jnp.einsum('hd,hde->he', kb_li.reshape(H, D), a_rel_lj)
    bv_rel = jnp.einsum('hd,hde->he', vb_li.reshape(H, D), m_rel_lj)
    Wcat = jnp.stack([Wk_rel, Wv_rel], axis=1)
    Wcat = Wcat.transpose(2, 0, 1, 3).reshape(HID, H * 2 * D)
    bcat = jnp.stack([bk_rel, bv_rel], axis=1).reshape(H * 2 * D)
    return Wcat, bcat


def _pad_to(n, unit):
    return ((n + unit - 1) // unit) * unit


def kernel(x_bus, x_gen, lin_W, lin_b, kW, kb, qW, qb, vW, vb, aW, ab, skip,
           a_rel, m_rel, p_rel, fW, fb, bn_g, bn_b, fW_out, fb_out,
           edge_index_bb, edge_index_gb, edge_index_bg, node_idx, num_graphs):
    n_bus = x_bus.shape[0]
    n_gen = x_gen.shape[0]
    nacc_b = _pad_to(n_bus + 16, 2048)
    nacc_g = _pad_to(n_gen + 16, 2048)

    xb = _matmul(x_bus, lin_W[0], lin_b[0], relu=True)
    xg = _matmul(x_gen, lin_W[1], lin_b[1], relu=True)
    x = {'bus': xb, 'gen': xg}
    sk_all = jax.nn.sigmoid(skip)
    edges = [('bus', 'bus', edge_index_bb), ('gen', 'bus', edge_index_gb),
             ('bus', 'gen', edge_index_bg)]
    ntype = {'bus': 0, 'gen': 1}
    naccs = {'bus': nacc_b, 'gen': nacc_g}

    rowps, colps, nwins = {}, {}, {}
    for j, (src, dst, ei) in enumerate(edges):
        E = ei.shape[1]
        E_pad = _pad_to(E, 2 * NW * WE)
        nwins[j] = E_pad // (NW * WE)
        npad = E_pad - E
        ar = jnp.arange(npad, dtype=jnp.int32)
        n_dst = n_bus if dst == 'bus' else n_gen
        rowps[j] = jnp.concatenate([ei[0].astype(jnp.int32), ar % 64])
        colps[j] = jnp.concatenate([ei[1].astype(jnp.int32), n_dst + ar % 16])

    for l in range(2):
        Wkv, bkv = {}, {}
        for j in range(3):
            i = ntype[edges[j][0]]
            Wkv[j], bkv[j] = _fold_kv(kW[l, i], kb[l, i], vW[l, i], vb[l, i],
                                      a_rel[l, j], m_rel[l, j])
        W_bus = jnp.concatenate([qW[l, 0], Wkv[0], Wkv[2]], axis=1)
        b_bus = jnp.concatenate([qb[l, 0], bkv[0], bkv[2]])
        W_gen = jnp.concatenate([qW[l, 1], Wkv[1]], axis=1)
        b_gen = jnp.concatenate([qb[l, 1], bkv[1]])
        P_bus = _matmul(x['bus'], W_bus, b_bus)
        P_gen = _matmul(x['gen'], W_gen, b_gen)
        qpad = {
            'bus': jnp.pad(P_bus[:, :HID], ((0, nacc_b - n_bus), (0, 0))),
            'gen': jnp.pad(P_gen[:, :HID], ((0, nacc_g - n_gen), (0, 0))),
        }
        kv = {0: P_bus[:, HID:HID + 2 * HID], 1: P_gen[:, HID:],
              2: P_bus[:, HID + 2 * HID:]}

        acc = {'bus': [], 'gen': []}
        for j, (src, dst, ei) in enumerate(edges):
            psc = jnp.zeros((16,), jnp.float32).at[:H].set(p_rel[l, j] / SQRT_D)
            A = _edge_sc(qpad[dst], kv[j], rowps[j], colps[j], psc,
                         naccs[dst], nwins[j])
            acc[dst].append(A)
        newx = {}
        for nt in ('bus', 'gen'):
            i = ntype[nt]
            newx[nt] = _combine(acc[nt], x[nt], aW[l, i], ab[l, i],
                                sk_all[l, i])
        x = newx

    G = n_bus // 19
    idxs = (node_idx[0][None, :] + 19 * jnp.arange(G)[:, None]).reshape(-1)
    n_sel = idxs.shape[0]
    n_sel_pad = _pad_to(n_sel, NW * W)
    ar = jnp.arange(n_sel_pad - n_sel, dtype=jnp.int32)
    idx_pad = jnp.concatenate([idxs.astype(jnp.int32), ar % 64])
    h = _gather_rows(x['bus'], idx_pad)[:n_sel]
    return _mlp(h, fW, fb, bn_g, bn_b, fW_out, fb_out)
